# baseline (speedup 1.0000x reference)
import functools

import jax
import jax.numpy as jnp
from jax import lax
from jax.experimental import pallas as pl
from jax.experimental.pallas import tpu as pltpu
from jax.experimental.pallas import tpu_sc as plsc

N = 10000
E = 160000
D = 256
H = 128
N_SUB = 16
BLK = 128
EPAD_ROWS = 1280
N_ACC = N + 16
SINK = N

ROW_CHUNK = 640
LAST_CHUNK = N - 15 * ROW_CHUNK
ZROWS = 128

MAIN_RPT = EPAD_ROWS // N_SUB
MAIN_PHASES = 2
MAIN_RPP = MAIN_RPT // MAIN_PHASES
DEG_RPC = EPAD_ROWS // 2
DEG_RPT = DEG_RPC // N_SUB
DEG_INFLIGHT = 4

_MESH = plsc.VectorSubcoreMesh(core_axis_name="c", subcore_axis_name="s")


def _tile_row0(tile):
    return pl.multiple_of(tile * ROW_CHUNK, 8)


def _zero_acc_slice(acc, zbuf, tile):
    nz = zbuf.shape[0]
    last_main = (LAST_CHUNK // nz) * nz
    tail = LAST_CHUNK - last_main
    row0 = _tile_row0(tile)

    @pl.when(tile < 15)
    def _():
        @pl.loop(0, ROW_CHUNK, step=nz)
        def _(m):
            pltpu.sync_copy(zbuf, acc.at[pl.ds(pl.multiple_of(row0 + m, 8), nz)])

    @pl.when(tile == 15)
    def _():
        @pl.loop(0, last_main, step=nz)
        def _(m):
            pltpu.sync_copy(zbuf, acc.at[pl.ds(pl.multiple_of(row0 + m, 8), nz)])

        if tail:
            pltpu.sync_copy(
                zbuf.at[pl.ds(0, tail)],
                acc.at[pl.ds(pl.multiple_of(row0 + last_main, 8), tail)])


def _copy_out_slice(acc, out_hbm, tile):
    row0 = _tile_row0(tile)

    @pl.when(tile < 15)
    def _():
        pltpu.sync_copy(acc.at[pl.ds(row0, ROW_CHUNK)],
                        out_hbm.at[pl.ds(row0, ROW_CHUNK)])

    @pl.when(tile == 15)
    def _():
        pltpu.sync_copy(acc.at[pl.ds(row0, LAST_CHUNK)],
                        out_hbm.at[pl.ds(row0, LAST_CHUNK)])


DEG_W = H


def _sc_degree(dst2, ones_hbm, zeros_hbm):
    @functools.partial(
        pl.kernel,
        out_type=(
            jax.ShapeDtypeStruct((N, DEG_W), jnp.float32),
            jax.ShapeDtypeStruct((N, DEG_W), jnp.float32),
        ),
        mesh=_MESH,
        scratch_types=[
            pltpu.VMEM_SHARED((N_ACC, DEG_W), jnp.float32),
            pltpu.VMEM((DEG_RPT, BLK), jnp.int32),
            pltpu.VMEM((BLK, DEG_W), jnp.float32),
            pltpu.VMEM((ZROWS, DEG_W), jnp.float32),
            pltpu.SemaphoreType.DMA,
        ],
    )
    def deg_kernel(dst2_hbm, ones_in, zeros_in, dega_hbm, degb_hbm,
                   acc, idx, ones_buf, zbuf, sem):
        c = lax.axis_index("c")
        t = lax.axis_index("s")

        pltpu.sync_copy(zeros_in, zbuf)
        _zero_acc_slice(acc, zbuf, t)
        pltpu.sync_copy(ones_in, ones_buf)
        irow0 = pl.multiple_of(c * DEG_RPC + t * DEG_RPT, 8)
        pltpu.sync_copy(dst2_hbm.at[pl.ds(irow0, DEG_RPT)], idx)

        plsc.subcore_barrier()

        @pl.loop(0, DEG_RPT)
        def _(k):
            pltpu.async_copy(ones_buf, acc.at[idx.at[k]], sem, add=True)

            @pl.when(k >= DEG_INFLIGHT)
            def _():
                pltpu.make_async_copy(
                    ones_buf, acc.at[idx.at[k - DEG_INFLIGHT]], sem).wait()

        for i in range(DEG_INFLIGHT):
            pltpu.make_async_copy(
                ones_buf, acc.at[idx.at[DEG_RPT - DEG_INFLIGHT + i]], sem).wait()

        plsc.subcore_barrier()

        @pl.when(c == 0)
        def _():
            _copy_out_slice(acc, dega_hbm, t)

        @pl.when(c == 1)
        def _():
            _copy_out_slice(acc, degb_hbm, t)

    return deg_kernel(dst2, ones_hbm, zeros_hbm)


SC_E_PER_TILE = E // N_SUB
SC_BLK = 40
SC_NBLK = SC_E_PER_TILE // SC_BLK


def _sc_scatter(ga, gb, src1, dst1, zeros_hbm):
    @functools.partial(
        pl.kernel,
        out_type=(
            jax.ShapeDtypeStruct((N, H), jnp.float32),
            jax.ShapeDtypeStruct((N, H), jnp.float32),
        ),
        mesh=_MESH,
        scratch_types=[
            pltpu.VMEM_SHARED((N_ACC, H), jnp.float32),
            pltpu.VMEM((SC_E_PER_TILE,), jnp.int32),
            pltpu.VMEM((SC_E_PER_TILE,), jnp.int32),
            pltpu.VMEM((SC_BLK,), jnp.int32),
            pltpu.VMEM((SC_BLK,), jnp.int32),
            pltpu.VMEM((SC_BLK, H), jnp.float32),
            pltpu.VMEM((SC_BLK, H), jnp.float32),
            pltpu.SemaphoreType.DMA,
            pltpu.SemaphoreType.DMA,
        ],
    )
    def scat_kernel(ga_hbm, gb_hbm, src_hbm, dst_hbm, zeros_in, sa_hbm, sb_hbm,
                    acc, src_all, dst_all, di0, di1, rows0, rows1, gsem0, gsem1):
        c = lax.axis_index("c")
        t = lax.axis_index("s")

        pltpu.sync_copy(zeros_in.at[pl.ds(0, SC_BLK)], rows0)
        _zero_acc_slice(acc, rows0, t)

        ebase = t * SC_E_PER_TILE
        pltpu.sync_copy(src_hbm.at[pl.ds(ebase, SC_E_PER_TILE)], src_all)
        pltpu.sync_copy(dst_hbm.at[pl.ds(ebase, SC_E_PER_TILE)], dst_all)

        plsc.subcore_barrier()

        def run(g_hbm, out_hbm):
            slots = ((rows0, di0, gsem0), (rows1, di1, gsem1))

            def gidx(kk):
                return src_all.at[pl.ds(kk * SC_BLK, SC_BLK)]

            def do_block(kk, rb, di, gsem, last):
                @pl.loop(0, SC_BLK, step=16)
                def _(j):
                    di.at[pl.ds(j, 16)][...] = dst_all.at[
                        pl.ds(kk * SC_BLK + j, 16)][...]
                pltpu.make_async_copy(g_hbm.at[gidx(kk)], rb, gsem).wait()
                pltpu.sync_copy(rb, acc.at[di], add=True)
                if not last:
                    @pl.when(kk + 2 < SC_NBLK)
                    def _():
                        pltpu.async_copy(g_hbm.at[gidx(kk + 2)], rb, gsem)

            for s, (rb, _di, gsem) in enumerate(slots):
                pltpu.async_copy(g_hbm.at[gidx(s)], rb, gsem)

            paired = SC_NBLK - (SC_NBLK % 2)

            @pl.loop(0, paired, step=2)
            def _(k):
                for s, (rb, di, gsem) in enumerate(slots):
                    do_block(k + s, rb, di, gsem, last=False)

            if SC_NBLK % 2:
                do_block(SC_NBLK - 1, rows0, di0, gsem0, last=True)

            plsc.subcore_barrier()
            _copy_out_slice(acc, out_hbm, t)

        @pl.when(c == 0)
        def _():
            run(ga_hbm, sa_hbm)

        @pl.when(c == 1)
        def _():
            run(gb_hbm, sb_hbm)

    return scat_kernel(ga, gb, src1, dst1, zeros_hbm)


def _tc_matmul_scale(x, w, dega, degb):
    def body(x_ref, w_ref, da_ref, db_ref, ga_ref, gb_ref, dinv_ref):
        h = jnp.dot(x_ref[...], w_ref[...], preferred_element_type=jnp.float32)
        deg = 1.0 + da_ref[:, 0:1] + db_ref[:, 0:1]
        dinv = lax.rsqrt(deg)
        g = h * dinv
        ga_ref[...] = g[:, :H]
        gb_ref[...] = g[:, H:]
        dinv_ref[...] = jnp.broadcast_to(dinv, (dinv.shape[0], 8))

    return pl.pallas_call(
        body,
        grid=(10,),
        in_specs=[
            pl.BlockSpec((1000, D), lambda i: (i, 0)),
            pl.BlockSpec((D, D), lambda i: (0, 0)),
            pl.BlockSpec((1000, DEG_W), lambda i: (i, 0)),
            pl.BlockSpec((1000, DEG_W), lambda i: (i, 0)),
        ],
        out_specs=[
            pl.BlockSpec((1000, H), lambda i: (i, 0)),
            pl.BlockSpec((1000, H), lambda i: (i, 0)),
            pl.BlockSpec((1000, 8), lambda i: (i, 0)),
        ],
        out_shape=[
            jax.ShapeDtypeStruct((N, H), jnp.float32),
            jax.ShapeDtypeStruct((N, H), jnp.float32),
            jax.ShapeDtypeStruct((N, 8), jnp.float32),
        ],
    )(x, w, dega, degb)


def _tc_mid(sa, sb, ga, gb, dinv8, b0, w1):
    b0r = b0.reshape(1, D)

    def body(sa_ref, sb_ref, ga_ref, gb_ref, dv_ref, b0_ref, w1_ref,
             g1a_ref, g1b_ref):
        dinv = dv_ref[:, 0:1]
        pre_a = (sa_ref[...] + ga_ref[...]) * dinv
        pre_b = (sb_ref[...] + gb_ref[...]) * dinv
        pre = jnp.concatenate([pre_a, pre_b], axis=1) + b0_ref[...]
        a = jnp.tanh(pre)
        h1 = jnp.dot(a, w1_ref[...], preferred_element_type=jnp.float32)
        g1 = h1 * dinv
        g1a_ref[...] = g1[:, :H]
        g1b_ref[...] = g1[:, H:]

    return pl.pallas_call(
        body,
        grid=(10,),
        in_specs=[
            pl.BlockSpec((1000, H), lambda i: (i, 0)),
            pl.BlockSpec((1000, H), lambda i: (i, 0)),
            pl.BlockSpec((1000, H), lambda i: (i, 0)),
            pl.BlockSpec((1000, H), lambda i: (i, 0)),
            pl.BlockSpec((1000, 8), lambda i: (i, 0)),
            pl.BlockSpec((1, D), lambda i: (0, 0)),
            pl.BlockSpec((D, D), lambda i: (0, 0)),
        ],
        out_specs=[
            pl.BlockSpec((1000, H), lambda i: (i, 0)),
            pl.BlockSpec((1000, H), lambda i: (i, 0)),
        ],
        out_shape=[
            jax.ShapeDtypeStruct((N, H), jnp.float32),
            jax.ShapeDtypeStruct((N, H), jnp.float32),
        ],
    )(sa, sb, ga, gb, dinv8, b0r, w1)


def _tc_out(s1a, s1b, g1a, g1b, dinv8, b1):
    b1r = b1.reshape(1, D)

    def body(sa_ref, sb_ref, ga_ref, gb_ref, dv_ref, b1_ref, o_ref):
        dinv = dv_ref[:, 0:1]
        oa = (sa_ref[...] + ga_ref[...]) * dinv
        ob = (sb_ref[...] + gb_ref[...]) * dinv
        o_ref[...] = jnp.concatenate([oa, ob], axis=1) + b1_ref[...]

    return pl.pallas_call(
        body,
        grid=(10,),
        in_specs=[
            pl.BlockSpec((1000, H), lambda i: (i, 0)),
            pl.BlockSpec((1000, H), lambda i: (i, 0)),
            pl.BlockSpec((1000, H), lambda i: (i, 0)),
            pl.BlockSpec((1000, H), lambda i: (i, 0)),
            pl.BlockSpec((1000, 8), lambda i: (i, 0)),
            pl.BlockSpec((1, D), lambda i: (0, 0)),
        ],
        out_specs=pl.BlockSpec((1000, D), lambda i: (i, 0)),
        out_shape=jax.ShapeDtypeStruct((N, D), jnp.float32),
    )(s1a, s1b, g1a, g1b, dinv8, b1r)


def kernel(x, edge_index, x_batch, W0, b0, W1, b1):
    del x_batch
    npad = EPAD_ROWS * BLK - E
    src1 = edge_index[0]
    dst1 = edge_index[1]
    dst2 = jnp.concatenate(
        [dst1, jnp.full((npad,), SINK, jnp.int32)]).reshape(EPAD_ROWS, BLK)
    ones_deg = jnp.ones((BLK, DEG_W), jnp.float32)
    zeros_deg = jnp.zeros((ZROWS, DEG_W), jnp.float32)
    zeros_hbm = jnp.zeros((ZROWS, H), jnp.float32)

    dega, degb = _sc_degree(dst2, ones_deg, zeros_deg)
    ga, gb, dinv8 = _tc_matmul_scale(x, W0, dega, degb)
    sa, sb = _sc_scatter(ga, gb, src1, dst1, zeros_hbm)
    g1a, g1b = _tc_mid(sa, sb, ga, gb, dinv8, b0, W1)
    s1a, s1b = _sc_scatter(g1a, g1b, src1, dst1, zeros_hbm)
    return _tc_out(s1a, s1b, g1a, g1b, dinv8, b1)


# speedup vs baseline: 1.2262x; 1.2262x over previous
import functools

import jax
import jax.numpy as jnp
from jax import lax
from jax.experimental import pallas as pl
from jax.experimental.pallas import tpu as pltpu
from jax.experimental.pallas import tpu_sc as plsc

N = 10000
E = 160000
D = 256
H = 128
N_SUB = 16
BLK = 128
EPAD_ROWS = 1280
N_ACC = N + 16
SINK = N

ROW_CHUNK = 640
LAST_CHUNK = N - 15 * ROW_CHUNK
ZROWS = 128

MAIN_RPT = EPAD_ROWS // N_SUB
MAIN_PHASES = 2
MAIN_RPP = MAIN_RPT // MAIN_PHASES
DEG_RPC = EPAD_ROWS // 2
DEG_RPT = DEG_RPC // N_SUB
DEG_INFLIGHT = 4

_MESH = plsc.VectorSubcoreMesh(core_axis_name="c", subcore_axis_name="s")


def _tile_row0(tile):
    return pl.multiple_of(tile * ROW_CHUNK, 8)


def _zero_acc_slice(acc, zbuf, tile):
    nz = zbuf.shape[0]
    last_main = (LAST_CHUNK // nz) * nz
    tail = LAST_CHUNK - last_main
    row0 = _tile_row0(tile)

    @pl.when(tile < 15)
    def _():
        @pl.loop(0, ROW_CHUNK, step=nz)
        def _(m):
            pltpu.sync_copy(zbuf, acc.at[pl.ds(pl.multiple_of(row0 + m, 8), nz)])

    @pl.when(tile == 15)
    def _():
        @pl.loop(0, last_main, step=nz)
        def _(m):
            pltpu.sync_copy(zbuf, acc.at[pl.ds(pl.multiple_of(row0 + m, 8), nz)])

        if tail:
            pltpu.sync_copy(
                zbuf.at[pl.ds(0, tail)],
                acc.at[pl.ds(pl.multiple_of(row0 + last_main, 8), tail)])


def _copy_out_slice(acc, out_hbm, tile):
    row0 = _tile_row0(tile)

    @pl.when(tile < 15)
    def _():
        pltpu.sync_copy(acc.at[pl.ds(row0, ROW_CHUNK)],
                        out_hbm.at[pl.ds(row0, ROW_CHUNK)])

    @pl.when(tile == 15)
    def _():
        pltpu.sync_copy(acc.at[pl.ds(row0, LAST_CHUNK)],
                        out_hbm.at[pl.ds(row0, LAST_CHUNK)])


DEG_W = H


def _sc_degree(dst2, ones_hbm, zeros_hbm):
    @functools.partial(
        pl.kernel,
        out_type=(
            jax.ShapeDtypeStruct((N, DEG_W), jnp.float32),
            jax.ShapeDtypeStruct((N, DEG_W), jnp.float32),
        ),
        mesh=_MESH,
        scratch_types=[
            pltpu.VMEM_SHARED((N_ACC, DEG_W), jnp.float32),
            pltpu.VMEM((DEG_RPT, BLK), jnp.int32),
            pltpu.VMEM((BLK, DEG_W), jnp.float32),
            pltpu.VMEM((ZROWS, DEG_W), jnp.float32),
            pltpu.SemaphoreType.DMA,
        ],
    )
    def deg_kernel(dst2_hbm, ones_in, zeros_in, dega_hbm, degb_hbm,
                   acc, idx, ones_buf, zbuf, sem):
        c = lax.axis_index("c")
        t = lax.axis_index("s")

        pltpu.sync_copy(zeros_in, zbuf)
        _zero_acc_slice(acc, zbuf, t)
        pltpu.sync_copy(ones_in, ones_buf)
        irow0 = pl.multiple_of(c * DEG_RPC + t * DEG_RPT, 8)
        pltpu.sync_copy(dst2_hbm.at[pl.ds(irow0, DEG_RPT)], idx)

        plsc.subcore_barrier()

        @pl.loop(0, DEG_RPT)
        def _(k):
            pltpu.async_copy(ones_buf, acc.at[idx.at[k]], sem, add=True)

            @pl.when(k >= DEG_INFLIGHT)
            def _():
                pltpu.make_async_copy(
                    ones_buf, acc.at[idx.at[k - DEG_INFLIGHT]], sem).wait()

        for i in range(DEG_INFLIGHT):
            pltpu.make_async_copy(
                ones_buf, acc.at[idx.at[DEG_RPT - DEG_INFLIGHT + i]], sem).wait()

        plsc.subcore_barrier()

        @pl.when(c == 0)
        def _():
            _copy_out_slice(acc, dega_hbm, t)

        @pl.when(c == 1)
        def _():
            _copy_out_slice(acc, degb_hbm, t)

    return deg_kernel(dst2, ones_hbm, zeros_hbm)


SC_E_PER_TILE = E // N_SUB
SC_BLK = 80
SC_NBLK = SC_E_PER_TILE // SC_BLK


def _sc_scatter(ga, gb, src1, dst1, zeros_hbm):
    @functools.partial(
        pl.kernel,
        out_type=(
            jax.ShapeDtypeStruct((N, H), jnp.float32),
            jax.ShapeDtypeStruct((N, H), jnp.float32),
        ),
        mesh=_MESH,
        scratch_types=[
            pltpu.VMEM_SHARED((N_ACC, H), jnp.float32),
            pltpu.VMEM((SC_E_PER_TILE,), jnp.int32),
            pltpu.VMEM((SC_E_PER_TILE,), jnp.int32),
            pltpu.VMEM((SC_BLK,), jnp.int32),
            pltpu.VMEM((SC_BLK,), jnp.int32),
            pltpu.VMEM((SC_BLK, H), jnp.float32),
            pltpu.VMEM((SC_BLK, H), jnp.float32),
            pltpu.SemaphoreType.DMA,
            pltpu.SemaphoreType.DMA,
        ],
    )
    def scat_kernel(ga_hbm, gb_hbm, src_hbm, dst_hbm, zeros_in, sa_hbm, sb_hbm,
                    acc, src_all, dst_all, di0, di1, rows0, rows1, gsem0, gsem1):
        c = lax.axis_index("c")
        t = lax.axis_index("s")

        pltpu.sync_copy(zeros_in.at[pl.ds(0, SC_BLK)], rows0)
        _zero_acc_slice(acc, rows0, t)

        ebase = t * SC_E_PER_TILE
        pltpu.sync_copy(src_hbm.at[pl.ds(ebase, SC_E_PER_TILE)], src_all)
        pltpu.sync_copy(dst_hbm.at[pl.ds(ebase, SC_E_PER_TILE)], dst_all)

        plsc.subcore_barrier()

        def run(g_hbm, out_hbm):
            slots = ((rows0, di0, gsem0), (rows1, di1, gsem1))

            def gidx(kk):
                return src_all.at[pl.ds(kk * SC_BLK, SC_BLK)]

            def do_block(kk, rb, di, gsem, last):
                @pl.loop(0, SC_BLK, step=16)
                def _(j):
                    di.at[pl.ds(j, 16)][...] = dst_all.at[
                        pl.ds(kk * SC_BLK + j, 16)][...]
                pltpu.make_async_copy(g_hbm.at[gidx(kk)], rb, gsem).wait()
                pltpu.sync_copy(rb, acc.at[di], add=True)
                if not last:
                    @pl.when(kk + 2 < SC_NBLK)
                    def _():
                        pltpu.async_copy(g_hbm.at[gidx(kk + 2)], rb, gsem)

            for s, (rb, _di, gsem) in enumerate(slots):
                pltpu.async_copy(g_hbm.at[gidx(s)], rb, gsem)

            paired = SC_NBLK - (SC_NBLK % 2)

            @pl.loop(0, paired, step=2)
            def _(k):
                for s, (rb, di, gsem) in enumerate(slots):
                    do_block(k + s, rb, di, gsem, last=False)

            if SC_NBLK % 2:
                do_block(SC_NBLK - 1, rows0, di0, gsem0, last=True)

            plsc.subcore_barrier()
            _copy_out_slice(acc, out_hbm, t)

        @pl.when(c == 0)
        def _():
            run(ga_hbm, sa_hbm)

        @pl.when(c == 1)
        def _():
            run(gb_hbm, sb_hbm)

    return scat_kernel(ga, gb, src1, dst1, zeros_hbm)


def _tc_matmul_scale(x, w, dega, degb):
    def body(x_ref, w_ref, da_ref, db_ref, ga_ref, gb_ref, dinv_ref):
        h = jnp.dot(x_ref[...], w_ref[...], preferred_element_type=jnp.float32)
        deg = 1.0 + da_ref[:, 0:1] + db_ref[:, 0:1]
        dinv = lax.rsqrt(deg)
        g = h * dinv
        ga_ref[...] = g[:, :H]
        gb_ref[...] = g[:, H:]
        dinv_ref[...] = jnp.broadcast_to(dinv, (dinv.shape[0], 8))

    return pl.pallas_call(
        body,
        grid=(10,),
        in_specs=[
            pl.BlockSpec((1000, D), lambda i: (i, 0)),
            pl.BlockSpec((D, D), lambda i: (0, 0)),
            pl.BlockSpec((1000, DEG_W), lambda i: (i, 0)),
            pl.BlockSpec((1000, DEG_W), lambda i: (i, 0)),
        ],
        out_specs=[
            pl.BlockSpec((1000, H), lambda i: (i, 0)),
            pl.BlockSpec((1000, H), lambda i: (i, 0)),
            pl.BlockSpec((1000, 8), lambda i: (i, 0)),
        ],
        out_shape=[
            jax.ShapeDtypeStruct((N, H), jnp.float32),
            jax.ShapeDtypeStruct((N, H), jnp.float32),
            jax.ShapeDtypeStruct((N, 8), jnp.float32),
        ],
    )(x, w, dega, degb)


def _tc_mid(sa, sb, ga, gb, dinv8, b0, w1):
    b0r = b0.reshape(1, D)

    def body(sa_ref, sb_ref, ga_ref, gb_ref, dv_ref, b0_ref, w1_ref,
             g1a_ref, g1b_ref):
        dinv = dv_ref[:, 0:1]
        pre_a = (sa_ref[...] + ga_ref[...]) * dinv
        pre_b = (sb_ref[...] + gb_ref[...]) * dinv
        pre = jnp.concatenate([pre_a, pre_b], axis=1) + b0_ref[...]
        a = jnp.tanh(pre)
        h1 = jnp.dot(a, w1_ref[...], preferred_element_type=jnp.float32)
        g1 = h1 * dinv
        g1a_ref[...] = g1[:, :H]
        g1b_ref[...] = g1[:, H:]

    return pl.pallas_call(
        body,
        grid=(10,),
        in_specs=[
            pl.BlockSpec((1000, H), lambda i: (i, 0)),
            pl.BlockSpec((1000, H), lambda i: (i, 0)),
            pl.BlockSpec((1000, H), lambda i: (i, 0)),
            pl.BlockSpec((1000, H), lambda i: (i, 0)),
            pl.BlockSpec((1000, 8), lambda i: (i, 0)),
            pl.BlockSpec((1, D), lambda i: (0, 0)),
            pl.BlockSpec((D, D), lambda i: (0, 0)),
        ],
        out_specs=[
            pl.BlockSpec((1000, H), lambda i: (i, 0)),
            pl.BlockSpec((1000, H), lambda i: (i, 0)),
        ],
        out_shape=[
            jax.ShapeDtypeStruct((N, H), jnp.float32),
            jax.ShapeDtypeStruct((N, H), jnp.float32),
        ],
    )(sa, sb, ga, gb, dinv8, b0r, w1)


def _tc_out(s1a, s1b, g1a, g1b, dinv8, b1):
    b1r = b1.reshape(1, D)

    def body(sa_ref, sb_ref, ga_ref, gb_ref, dv_ref, b1_ref, o_ref):
        dinv = dv_ref[:, 0:1]
        oa = (sa_ref[...] + ga_ref[...]) * dinv
        ob = (sb_ref[...] + gb_ref[...]) * dinv
        o_ref[...] = jnp.concatenate([oa, ob], axis=1) + b1_ref[...]

    return pl.pallas_call(
        body,
        grid=(10,),
        in_specs=[
            pl.BlockSpec((1000, H), lambda i: (i, 0)),
            pl.BlockSpec((1000, H), lambda i: (i, 0)),
            pl.BlockSpec((1000, H), lambda i: (i, 0)),
            pl.BlockSpec((1000, H), lambda i: (i, 0)),
            pl.BlockSpec((1000, 8), lambda i: (i, 0)),
            pl.BlockSpec((1, D), lambda i: (0, 0)),
        ],
        out_specs=pl.BlockSpec((1000, D), lambda i: (i, 0)),
        out_shape=jax.ShapeDtypeStruct((N, D), jnp.float32),
    )(s1a, s1b, g1a, g1b, dinv8, b1r)


def kernel(x, edge_index, x_batch, W0, b0, W1, b1):
    del x_batch
    npad = EPAD_ROWS * BLK - E
    src1 = edge_index[0]
    dst1 = edge_index[1]
    dst2 = jnp.concatenate(
        [dst1, jnp.full((npad,), SINK, jnp.int32)]).reshape(EPAD_ROWS, BLK)
    ones_deg = jnp.ones((BLK, DEG_W), jnp.float32)
    zeros_deg = jnp.zeros((ZROWS, DEG_W), jnp.float32)
    zeros_hbm = jnp.zeros((ZROWS, H), jnp.float32)

    dega, degb = _sc_degree(dst2, ones_deg, zeros_deg)
    ga, gb, dinv8 = _tc_matmul_scale(x, W0, dega, degb)
    sa, sb = _sc_scatter(ga, gb, src1, dst1, zeros_hbm)
    g1a, g1b = _tc_mid(sa, sb, ga, gb, dinv8, b0, W1)
    s1a, s1b = _sc_scatter(g1a, g1b, src1, dst1, zeros_hbm)
    return _tc_out(s1a, s1b, g1a, g1b, dinv8, b1)


# speedup vs baseline: 1.2275x; 1.0010x over previous
import functools

import jax
import jax.numpy as jnp
from jax import lax
from jax.experimental import pallas as pl
from jax.experimental.pallas import tpu as pltpu
from jax.experimental.pallas import tpu_sc as plsc

N = 10000
E = 160000
D = 256
H = 128
N_SUB = 16
BLK = 128
EPAD_ROWS = 1280
N_ACC = N + 16
SINK = N

ROW_CHUNK = 640
LAST_CHUNK = N - 15 * ROW_CHUNK
ZROWS = 128

DEG_RPC = EPAD_ROWS // 2
DEG_RPT = DEG_RPC // N_SUB
DEG_INFLIGHT = 4

_MESH = plsc.VectorSubcoreMesh(core_axis_name="c", subcore_axis_name="s")


def _tile_row0(tile):
    return pl.multiple_of(tile * ROW_CHUNK, 8)


def _zero_acc_slice(acc, zbuf, tile):
    nz = zbuf.shape[0]
    last_main = (LAST_CHUNK // nz) * nz
    tail = LAST_CHUNK - last_main
    row0 = _tile_row0(tile)

    @pl.when(tile < 15)
    def _():
        @pl.loop(0, ROW_CHUNK, step=nz)
        def _(m):
            pltpu.sync_copy(zbuf, acc.at[pl.ds(pl.multiple_of(row0 + m, 8), nz)])

    @pl.when(tile == 15)
    def _():
        @pl.loop(0, last_main, step=nz)
        def _(m):
            pltpu.sync_copy(zbuf, acc.at[pl.ds(pl.multiple_of(row0 + m, 8), nz)])

        if tail:
            pltpu.sync_copy(
                zbuf.at[pl.ds(0, tail)],
                acc.at[pl.ds(pl.multiple_of(row0 + last_main, 8), tail)])


def _copy_out_slice(acc, out_hbm, tile):
    row0 = _tile_row0(tile)

    @pl.when(tile < 15)
    def _():
        pltpu.sync_copy(acc.at[pl.ds(row0, ROW_CHUNK)],
                        out_hbm.at[pl.ds(row0, ROW_CHUNK)])

    @pl.when(tile == 15)
    def _():
        pltpu.sync_copy(acc.at[pl.ds(row0, LAST_CHUNK)],
                        out_hbm.at[pl.ds(row0, LAST_CHUNK)])


DEG_W = H


def _sc_degree(dst2, ones_hbm, zeros_hbm):
    @functools.partial(
        pl.kernel,
        out_type=(
            jax.ShapeDtypeStruct((N, DEG_W), jnp.float32),
            jax.ShapeDtypeStruct((N, DEG_W), jnp.float32),
        ),
        mesh=_MESH,
        scratch_types=[
            pltpu.VMEM_SHARED((N_ACC, DEG_W), jnp.float32),
            pltpu.VMEM((DEG_RPT, BLK), jnp.int32),
            pltpu.VMEM((BLK, DEG_W), jnp.float32),
            pltpu.VMEM((ZROWS, DEG_W), jnp.float32),
            pltpu.SemaphoreType.DMA,
        ],
    )
    def deg_kernel(dst2_hbm, ones_in, zeros_in, dega_hbm, degb_hbm,
                   acc, idx, ones_buf, zbuf, sem):
        c = lax.axis_index("c")
        t = lax.axis_index("s")

        pltpu.sync_copy(zeros_in, zbuf)
        _zero_acc_slice(acc, zbuf, t)
        pltpu.sync_copy(ones_in, ones_buf)
        irow0 = pl.multiple_of(c * DEG_RPC + t * DEG_RPT, 8)
        pltpu.sync_copy(dst2_hbm.at[pl.ds(irow0, DEG_RPT)], idx)

        plsc.subcore_barrier()

        @pl.loop(0, DEG_RPT)
        def _(k):
            pltpu.async_copy(ones_buf, acc.at[idx.at[k]], sem, add=True)

            @pl.when(k >= DEG_INFLIGHT)
            def _():
                pltpu.make_async_copy(
                    ones_buf, acc.at[idx.at[k - DEG_INFLIGHT]], sem).wait()

        for i in range(DEG_INFLIGHT):
            pltpu.make_async_copy(
                ones_buf, acc.at[idx.at[DEG_RPT - DEG_INFLIGHT + i]], sem).wait()

        plsc.subcore_barrier()

        @pl.when(c == 0)
        def _():
            _copy_out_slice(acc, dega_hbm, t)

        @pl.when(c == 1)
        def _():
            _copy_out_slice(acc, degb_hbm, t)

    return deg_kernel(dst2, ones_hbm, zeros_hbm)


SC_E_PER_TILE = E // N_SUB
SC_BLK = 80
SC_NBLK = SC_E_PER_TILE // SC_BLK


def _sc_scatter(ga, gb, src1, dst1, zeros_hbm):
    @functools.partial(
        pl.kernel,
        out_type=(
            jax.ShapeDtypeStruct((N, H), jnp.float32),
            jax.ShapeDtypeStruct((N, H), jnp.float32),
        ),
        mesh=_MESH,
        scratch_types=[
            pltpu.VMEM_SHARED((N_ACC, H), jnp.float32),
            pltpu.VMEM((SC_E_PER_TILE,), jnp.int32),
            pltpu.VMEM((SC_E_PER_TILE,), jnp.int32),
            pltpu.VMEM((SC_BLK,), jnp.int32),
            pltpu.VMEM((SC_BLK,), jnp.int32),
            pltpu.VMEM((SC_BLK, H), jnp.float32),
            pltpu.VMEM((SC_BLK, H), jnp.float32),
            pltpu.SemaphoreType.DMA,
            pltpu.SemaphoreType.DMA,
        ],
    )
    def scat_kernel(ga_hbm, gb_hbm, src_hbm, dst_hbm, zeros_in, sa_hbm, sb_hbm,
                    acc, src_all, dst_all, di0, di1, rows0, rows1, gsem0, gsem1):
        c = lax.axis_index("c")
        t = lax.axis_index("s")

        pltpu.sync_copy(zeros_in.at[pl.ds(0, SC_BLK)], rows0)
        _zero_acc_slice(acc, rows0, t)

        ebase = t * SC_E_PER_TILE
        pltpu.sync_copy(src_hbm.at[pl.ds(ebase, SC_E_PER_TILE)], src_all)
        pltpu.sync_copy(dst_hbm.at[pl.ds(ebase, SC_E_PER_TILE)], dst_all)

        plsc.subcore_barrier()

        def run(g_hbm, out_hbm):
            slots = ((rows0, di0, gsem0), (rows1, di1, gsem1))

            def gidx(kk):
                return src_all.at[pl.ds(kk * SC_BLK, SC_BLK)]

            def do_block(kk, rb, di, gsem, last):
                @pl.loop(0, SC_BLK, step=16)
                def _(j):
                    di.at[pl.ds(j, 16)][...] = dst_all.at[
                        pl.ds(kk * SC_BLK + j, 16)][...]
                pltpu.make_async_copy(g_hbm.at[gidx(kk)], rb, gsem).wait()
                pltpu.sync_copy(rb, acc.at[di], add=True)
                if not last:
                    @pl.when(kk + 2 < SC_NBLK)
                    def _():
                        pltpu.async_copy(g_hbm.at[gidx(kk + 2)], rb, gsem)

            for s, (rb, _di, gsem) in enumerate(slots):
                pltpu.async_copy(g_hbm.at[gidx(s)], rb, gsem)

            paired = SC_NBLK - (SC_NBLK % 2)

            @pl.loop(0, paired, step=2)
            def _(k):
                for s, (rb, di, gsem) in enumerate(slots):
                    do_block(k + s, rb, di, gsem, last=False)

            if SC_NBLK % 2:
                do_block(SC_NBLK - 1, rows0, di0, gsem0, last=True)

            plsc.subcore_barrier()
            _copy_out_slice(acc, out_hbm, t)

        @pl.when(c == 0)
        def _():
            run(ga_hbm, sa_hbm)

        @pl.when(c == 1)
        def _():
            run(gb_hbm, sb_hbm)

    return scat_kernel(ga, gb, src1, dst1, zeros_hbm)


def _tc_matmul_scale(x, w, dega, degb):
    def body(x_ref, w_ref, da_ref, db_ref, ga_ref, gb_ref, dinv_ref):
        h = jnp.dot(x_ref[...], w_ref[...], preferred_element_type=jnp.float32)
        deg = 1.0 + da_ref[:, 0:1] + db_ref[:, 0:1]
        dinv = lax.rsqrt(deg)
        g = h * dinv
        ga_ref[...] = g[:, :H]
        gb_ref[...] = g[:, H:]
        dinv_ref[...] = jnp.broadcast_to(dinv, (dinv.shape[0], 8))

    return pl.pallas_call(
        body,
        grid=(10,),
        in_specs=[
            pl.BlockSpec((1000, D), lambda i: (i, 0)),
            pl.BlockSpec((D, D), lambda i: (0, 0)),
            pl.BlockSpec((1000, DEG_W), lambda i: (i, 0)),
            pl.BlockSpec((1000, DEG_W), lambda i: (i, 0)),
        ],
        out_specs=[
            pl.BlockSpec((1000, H), lambda i: (i, 0)),
            pl.BlockSpec((1000, H), lambda i: (i, 0)),
            pl.BlockSpec((1000, 8), lambda i: (i, 0)),
        ],
        out_shape=[
            jax.ShapeDtypeStruct((N, H), jnp.float32),
            jax.ShapeDtypeStruct((N, H), jnp.float32),
            jax.ShapeDtypeStruct((N, 8), jnp.float32),
        ],
    )(x, w, dega, degb)


def _tc_mid(sa, sb, ga, gb, dinv8, b0, w1):
    b0r = b0.reshape(1, D)

    def body(sa_ref, sb_ref, ga_ref, gb_ref, dv_ref, b0_ref, w1_ref,
             g1a_ref, g1b_ref):
        dinv = dv_ref[:, 0:1]
        pre_a = (sa_ref[...] + ga_ref[...]) * dinv
        pre_b = (sb_ref[...] + gb_ref[...]) * dinv
        pre = jnp.concatenate([pre_a, pre_b], axis=1) + b0_ref[...]
        a = jnp.tanh(pre)
        h1 = jnp.dot(a, w1_ref[...], preferred_element_type=jnp.float32)
        g1 = h1 * dinv
        g1a_ref[...] = g1[:, :H]
        g1b_ref[...] = g1[:, H:]

    return pl.pallas_call(
        body,
        grid=(10,),
        in_specs=[
            pl.BlockSpec((1000, H), lambda i: (i, 0)),
            pl.BlockSpec((1000, H), lambda i: (i, 0)),
            pl.BlockSpec((1000, H), lambda i: (i, 0)),
            pl.BlockSpec((1000, H), lambda i: (i, 0)),
            pl.BlockSpec((1000, 8), lambda i: (i, 0)),
            pl.BlockSpec((1, D), lambda i: (0, 0)),
            pl.BlockSpec((D, D), lambda i: (0, 0)),
        ],
        out_specs=[
            pl.BlockSpec((1000, H), lambda i: (i, 0)),
            pl.BlockSpec((1000, H), lambda i: (i, 0)),
        ],
        out_shape=[
            jax.ShapeDtypeStruct((N, H), jnp.float32),
            jax.ShapeDtypeStruct((N, H), jnp.float32),
        ],
    )(sa, sb, ga, gb, dinv8, b0r, w1)


def _tc_out(s1a, s1b, g1a, g1b, dinv8, b1):
    b1r = b1.reshape(1, D)

    def body(sa_ref, sb_ref, ga_ref, gb_ref, dv_ref, b1_ref, o_ref):
        dinv = dv_ref[:, 0:1]
        oa = (sa_ref[...] + ga_ref[...]) * dinv
        ob = (sb_ref[...] + gb_ref[...]) * dinv
        o_ref[...] = jnp.concatenate([oa, ob], axis=1) + b1_ref[...]

    return pl.pallas_call(
        body,
        grid=(10,),
        in_specs=[
            pl.BlockSpec((1000, H), lambda i: (i, 0)),
            pl.BlockSpec((1000, H), lambda i: (i, 0)),
            pl.BlockSpec((1000, H), lambda i: (i, 0)),
            pl.BlockSpec((1000, H), lambda i: (i, 0)),
            pl.BlockSpec((1000, 8), lambda i: (i, 0)),
            pl.BlockSpec((1, D), lambda i: (0, 0)),
        ],
        out_specs=pl.BlockSpec((1000, D), lambda i: (i, 0)),
        out_shape=jax.ShapeDtypeStruct((N, D), jnp.float32),
    )(s1a, s1b, g1a, g1b, dinv8, b1r)


def kernel(x, edge_index, x_batch, W0, b0, W1, b1):
    del x_batch
    npad = EPAD_ROWS * BLK - E
    src1 = edge_index[0]
    dst1 = edge_index[1]
    dst2 = jnp.concatenate(
        [dst1, jnp.full((npad,), SINK, jnp.int32)]).reshape(EPAD_ROWS, BLK)
    ones_deg = jnp.ones((BLK, DEG_W), jnp.float32)
    zeros_deg = jnp.zeros((ZROWS, DEG_W), jnp.float32)
    zeros_hbm = jnp.zeros((ZROWS, H), jnp.float32)

    dega, degb = _sc_degree(dst2, ones_deg, zeros_deg)
    ga, gb, dinv8 = _tc_matmul_scale(x, W0, dega, degb)
    sa, sb = _sc_scatter(ga, gb, src1, dst1, zeros_hbm)
    g1a, g1b = _tc_mid(sa, sb, ga, gb, dinv8, b0, W1)
    s1a, s1b = _sc_scatter(g1a, g1b, src1, dst1, zeros_hbm)
    return _tc_out(s1a, s1b, g1a, g1b, dinv8, b1)
